# Initial kernel scaffold; baseline (speedup 1.0000x reference)
import jax
import jax.numpy as jnp
from jax.experimental import pallas as pl


def kernel(x, embeddings, labels):
    raise NotImplementedError("write your pallas kernel here")


# speedup vs baseline: 1.9262x; 1.9262x over previous
import functools

import jax
import jax.numpy as jnp
from jax.experimental import pallas as pl
from jax.experimental.pallas import tpu as pltpu

Q = 1024
D = 16
KTOT = 100000
KB = 2048
NBLK = 49
KPAD = NBLK * KB
NN = 8
NEG = float("-inf")
BIGI = 2**30


def _topk_body(an_ref, bn_ref, idx_out, val_out):
    step = pl.program_id(0)

    @pl.when(step == 0)
    def _init():
        val_out[...] = jnp.full((Q, NN), NEG, jnp.float32)
        idx_out[...] = jnp.full((Q, NN), BIGI, jnp.int32)

    base = step * KB
    sims = jax.lax.dot_general(
        an_ref[...], bn_ref[...],
        (((1,), (1,)), ((), ())),
        preferred_element_type=jnp.float32,
    )
    col = base + jax.lax.broadcasted_iota(jnp.int32, (Q, KB), 1)
    sims = jnp.where(col < KTOT, sims, NEG)

    rv = val_out[...]
    ri = idx_out[...]
    w = sims
    nv = []
    ni = []
    for _ in range(NN):
        m1 = jnp.max(w, axis=1, keepdims=True)
        m2 = jnp.max(rv, axis=1, keepdims=True)
        m = jnp.maximum(m1, m2)
        i1 = jnp.min(jnp.where(w == m, col, BIGI), axis=1, keepdims=True)
        i2 = jnp.min(jnp.where(rv == m, ri, BIGI), axis=1, keepdims=True)
        imin = jnp.minimum(i1, i2)
        nv.append(m)
        ni.append(imin)
        w = jnp.where(col == imin, NEG, w)
        rv = jnp.where(ri == imin, NEG, rv)
    val_out[...] = jnp.concatenate(nv, axis=1)
    idx_out[...] = jnp.concatenate(ni, axis=1)


@functools.partial(jax.jit, static_argnames=("interpret",))
def _topk_call(an, bnp, interpret=False):
    idx, _ = pl.pallas_call(
        _topk_body,
        grid=(NBLK,),
        in_specs=[
            pl.BlockSpec((Q, D), lambda i: (0, 0)),
            pl.BlockSpec((KB, D), lambda i: (i, 0)),
        ],
        out_specs=[
            pl.BlockSpec((Q, NN), lambda i: (0, 0)),
            pl.BlockSpec((Q, NN), lambda i: (0, 0)),
        ],
        out_shape=[
            jax.ShapeDtypeStruct((Q, NN), jnp.int32),
            jax.ShapeDtypeStruct((Q, NN), jnp.float32),
        ],
        interpret=interpret,
    )(an, bnp)
    return idx


def kernel(x, embeddings, labels):
    eps = 1e-8
    an = x / jnp.maximum(jnp.linalg.norm(x, axis=-1, keepdims=True), eps)
    bn = embeddings / jnp.maximum(
        jnp.linalg.norm(embeddings, axis=-1, keepdims=True), eps)
    bnp = jnp.pad(bn, ((0, KPAD - KTOT), (0, 0)))
    neighbors = _topk_call(an, bnp)
    neighbor_labels = labels[neighbors]
    counts = jnp.sum(
        jax.nn.one_hot(neighbor_labels, 1000, dtype=jnp.int32), axis=1)
    pred = jnp.argmax(counts, axis=1).astype(labels.dtype)
    return pred


# speedup vs baseline: 1.9338x; 1.0040x over previous
import functools

import jax
import jax.numpy as jnp
from jax.experimental import pallas as pl
from jax.experimental.pallas import tpu as pltpu
from jax.experimental.pallas import tpu_sc as plsc

Q = 1024
D = 16
KTOT = 100000
KB = 2048
NBLK = 49
KPAD = NBLK * KB
NN = 8
NEG = float("-inf")
BIGI = 2**30


def _topk_body(an_ref, bn_ref, idx_out, val_out):
    step = pl.program_id(0)

    @pl.when(step == 0)
    def _init():
        val_out[...] = jnp.full((Q, NN), NEG, jnp.float32)
        idx_out[...] = jnp.full((Q, NN), BIGI, jnp.int32)

    base = step * KB
    sims = jax.lax.dot_general(
        an_ref[...], bn_ref[...],
        (((1,), (1,)), ((), ())),
        preferred_element_type=jnp.float32,
    )
    col = base + jax.lax.broadcasted_iota(jnp.int32, (Q, KB), 1)
    sims = jnp.where(col < KTOT, sims, NEG)

    rv = val_out[...]
    ri = idx_out[...]
    w = sims
    nv = []
    ni = []
    for _ in range(NN):
        m1 = jnp.max(w, axis=1, keepdims=True)
        m2 = jnp.max(rv, axis=1, keepdims=True)
        m = jnp.maximum(m1, m2)
        i1 = jnp.min(jnp.where(w == m, col, BIGI), axis=1, keepdims=True)
        i2 = jnp.min(jnp.where(rv == m, ri, BIGI), axis=1, keepdims=True)
        imin = jnp.minimum(i1, i2)
        nv.append(m)
        ni.append(imin)
        w = jnp.where(col == imin, NEG, w)
        rv = jnp.where(ri == imin, NEG, rv)
    val_out[...] = jnp.concatenate(nv, axis=1)
    idx_out[...] = jnp.concatenate(ni, axis=1)


@functools.partial(jax.jit, static_argnames=("interpret",))
def _topk_call(an, bnp, interpret=False):
    idx, _ = pl.pallas_call(
        _topk_body,
        grid=(NBLK,),
        in_specs=[
            pl.BlockSpec((Q, D), lambda i: (0, 0)),
            pl.BlockSpec((KB, D), lambda i: (i, 0)),
        ],
        out_specs=[
            pl.BlockSpec((Q, NN), lambda i: (0, 0)),
            pl.BlockSpec((Q, NN), lambda i: (0, 0)),
        ],
        out_shape=[
            jax.ShapeDtypeStruct((Q, NN), jnp.int32),
            jax.ShapeDtypeStruct((Q, NN), jnp.float32),
        ],
        interpret=interpret,
    )(an, bnp)
    return idx


NW = 32
QPW = Q // NW
NGRP = QPW // 16


def _mode_body(arr_hbm, labels_hbm, out_hbm, idx_v, lab_v, pred_v, sem):
    wid = jax.lax.axis_index("s") * 2 + jax.lax.axis_index("c")
    pltpu.sync_copy(arr_hbm.at[wid], idx_v)
    copies = [
        pltpu.async_copy(labels_hbm.at[idx_v.at[r]], lab_v.at[r], sem)
        for r in range(16)
    ]
    for c in copies:
        c.wait()
    for g in range(NGRP):
        labs = [lab_v[j * NGRP + g, :] for j in range(NN)]
        best = jnp.full((16,), -1, jnp.int32)
        for i in range(NN):
            cnt = jnp.full((16,), 0, jnp.int32)
            for j in range(NN):
                cnt += jnp.where(labs[i] == labs[j], 1, 0)
            score = cnt * 2048 + (2047 - labs[i])
            best = jnp.maximum(best, score)
        pred_v[pl.ds(g * 16, 16)] = 2047 - (best & 2047)
    pltpu.sync_copy(pred_v, out_hbm.at[pl.ds(wid * QPW, QPW)])


@jax.jit
def _mode_call(arr, labels):
    mesh = plsc.VectorSubcoreMesh(core_axis_name="c", subcore_axis_name="s")
    f = pl.kernel(
        _mode_body,
        out_type=jax.ShapeDtypeStruct((Q,), jnp.int32),
        mesh=mesh,
        scratch_types=[
            pltpu.VMEM((16, 16), jnp.int32),
            pltpu.VMEM((16, 16), jnp.int32),
            pltpu.VMEM((QPW,), jnp.int32),
            pltpu.SemaphoreType.DMA,
        ],
    )
    return f(arr, labels)


def kernel(x, embeddings, labels):
    eps = 1e-8
    an = x / jnp.maximum(jnp.linalg.norm(x, axis=-1, keepdims=True), eps)
    bn = embeddings / jnp.maximum(
        jnp.linalg.norm(embeddings, axis=-1, keepdims=True), eps)
    bnp = jnp.pad(bn, ((0, KPAD - KTOT), (0, 0)))
    neighbors = _topk_call(an, bnp)
    arr = (neighbors.reshape(NW, NGRP, 16, NN)
           .transpose(0, 3, 1, 2).reshape(NW, NN * NGRP, 16))
    return _mode_call(arr, labels)


# speedup vs baseline: 3.1605x; 1.6343x over previous
import functools

import jax
import jax.numpy as jnp
from jax.experimental import pallas as pl
from jax.experimental.pallas import tpu as pltpu
from jax.experimental.pallas import tpu_sc as plsc

Q = 1024
D = 16
KTOT = 100000
KB = 2048
NBLK = 49
KPAD = NBLK * KB
NN = 8
NEG = float("-inf")
BIGI = 2**30


def _topk_body(an_ref, bn_ref, idx_out, val_out, w_ref):
    step = pl.program_id(0)

    @pl.when(step == 0)
    def _init():
        val_out[...] = jnp.full((NN, Q), NEG, jnp.float32)
        idx_out[...] = jnp.full((NN, Q), BIGI, jnp.int32)

    base = step * KB
    sims = jax.lax.dot_general(
        bn_ref[...], an_ref[...],
        (((1,), (1,)), ((), ())),
        preferred_element_type=jnp.float32,
    )
    colt = base + jax.lax.broadcasted_iota(jnp.int32, (KB, Q), 0)
    w_ref[...] = jnp.where(colt < KTOT, sims, NEG)

    def cond(go):
        return go

    def body(_go):
        w = w_ref[...]
        rv = val_out[...]
        ri = idx_out[...]
        t = rv[NN - 1:NN, :]
        m = jnp.max(w, axis=0, keepdims=True)
        imin = jnp.min(jnp.where(w == m, colt, BIGI), axis=0, keepdims=True)
        w_ref[...] = jnp.where(colt == imin, NEG, w)
        ge = [m > rv[s:s + 1, :] for s in range(NN)]
        nv, ni = [], []
        for s in range(NN):
            if s == 0:
                nv.append(jnp.where(ge[0], m, rv[0:1, :]))
                ni.append(jnp.where(ge[0], imin, ri[0:1, :]))
            else:
                nv.append(jnp.where(
                    ge[s], jnp.where(ge[s - 1], rv[s - 1:s, :], m),
                    rv[s:s + 1, :]))
                ni.append(jnp.where(
                    ge[s], jnp.where(ge[s - 1], ri[s - 1:s, :], imin),
                    ri[s:s + 1, :]))
        val_out[...] = jnp.concatenate(nv, axis=0)
        idx_out[...] = jnp.concatenate(ni, axis=0)
        return jnp.any(m > t)

    jax.lax.while_loop(cond, body, True)


@functools.partial(jax.jit, static_argnames=("interpret",))
def _topk_call(an, bnp, interpret=False):
    idx, _ = pl.pallas_call(
        _topk_body,
        grid=(NBLK,),
        in_specs=[
            pl.BlockSpec((Q, D), lambda i: (0, 0)),
            pl.BlockSpec((KB, D), lambda i: (i, 0)),
        ],
        out_specs=[
            pl.BlockSpec((NN, Q), lambda i: (0, 0)),
            pl.BlockSpec((NN, Q), lambda i: (0, 0)),
        ],
        out_shape=[
            jax.ShapeDtypeStruct((NN, Q), jnp.int32),
            jax.ShapeDtypeStruct((NN, Q), jnp.float32),
        ],
        scratch_shapes=[pltpu.VMEM((KB, Q), jnp.float32)],
        interpret=interpret,
    )(an, bnp)
    return idx


NW = 32
QPW = Q // NW
NGRP = QPW // 16


def _mode_body(arr_hbm, labels_hbm, out_hbm, idx_v, lab_v, pred_v, sem):
    wid = jax.lax.axis_index("s") * 2 + jax.lax.axis_index("c")
    pltpu.sync_copy(arr_hbm.at[wid], idx_v)
    copies = [
        pltpu.async_copy(labels_hbm.at[idx_v.at[r]], lab_v.at[r], sem)
        for r in range(16)
    ]
    for c in copies:
        c.wait()
    for g in range(NGRP):
        labs = [lab_v[j * NGRP + g, :] for j in range(NN)]
        best = jnp.full((16,), -1, jnp.int32)
        for i in range(NN):
            cnt = jnp.full((16,), 0, jnp.int32)
            for j in range(NN):
                cnt += jnp.where(labs[i] == labs[j], 1, 0)
            score = cnt * 2048 + (2047 - labs[i])
            best = jnp.maximum(best, score)
        pred_v[pl.ds(g * 16, 16)] = 2047 - (best & 2047)
    pltpu.sync_copy(pred_v, out_hbm.at[pl.ds(wid * QPW, QPW)])


@jax.jit
def _mode_call(arr, labels):
    mesh = plsc.VectorSubcoreMesh(core_axis_name="c", subcore_axis_name="s")
    f = pl.kernel(
        _mode_body,
        out_type=jax.ShapeDtypeStruct((Q,), jnp.int32),
        mesh=mesh,
        scratch_types=[
            pltpu.VMEM((16, 16), jnp.int32),
            pltpu.VMEM((16, 16), jnp.int32),
            pltpu.VMEM((QPW,), jnp.int32),
            pltpu.SemaphoreType.DMA,
        ],
    )
    return f(arr, labels)


def kernel(x, embeddings, labels):
    eps = 1e-8
    an = x / jnp.maximum(jnp.linalg.norm(x, axis=-1, keepdims=True), eps)
    bn = embeddings / jnp.maximum(
        jnp.linalg.norm(embeddings, axis=-1, keepdims=True), eps)
    bnp = jnp.pad(bn, ((0, KPAD - KTOT), (0, 0)))
    neighbors_t = _topk_call(an, bnp)
    arr = (neighbors_t.reshape(NN, NW, NGRP, 16)
           .transpose(1, 0, 2, 3).reshape(NW, NN * NGRP, 16))
    return _mode_call(arr, labels)
